# baseline (speedup 1.0000x reference)
import jax
import jax.numpy as jnp
from jax.experimental import pallas as pl
from jax.experimental.pallas import tpu as pltpu

B, C, H, W_ = 8, 96, 32, 32
N = H * W_
K = 9
G = 4
CG = C // G
EPS_GN = 1e-5
NEG = -3.0e38


def _gcn_block_kernel(x_ref, w1_ref, b1_ref, w2_ref, b2_ref,
                      g1w_ref, g1b_ref, g2w_ref, g2b_ref, out_ref):
    xc = x_ref[0]

    nrm = jnp.sqrt(jnp.sum(xc * xc, axis=0, keepdims=True))
    xn = xc / jnp.maximum(nrm, 1e-12)

    sim = jax.lax.dot_general(
        xn, xn, dimension_numbers=(((0,), (0,)), ((), ())),
        preferred_element_type=jnp.float32)

    m = jnp.max(sim, axis=0, keepdims=True)
    deg = m
    for _ in range(K - 1):
        m = jnp.max(jnp.where(sim < m, sim, NEG), axis=0, keepdims=True)
        deg = deg + m
    thresh = m
    deg = deg + 1e-6

    w = jnp.where(sim >= thresh, sim, 0.0)

    c_io = jax.lax.broadcasted_iota(jnp.int32, (G, C), 1)
    g_io = jax.lax.broadcasted_iota(jnp.int32, (G, C), 0)
    mt = (c_io // CG == g_io).astype(jnp.float32)

    def gcn_gn_silu(hc_in, w_ref, b_ref, gw_ref, gb_ref):
        xt = jax.lax.dot_general(
            w_ref[...], hc_in, dimension_numbers=(((0,), (0,)), ((), ())),
            preferred_element_type=jnp.float32)
        h = jnp.dot(xt, w, preferred_element_type=jnp.float32) / deg \
            + b_ref[...]
        s = jnp.sum(h, axis=1, keepdims=True)
        ss = jnp.sum(h * h, axis=1, keepdims=True)
        gs = jnp.dot(mt, s, preferred_element_type=jnp.float32)
        gss = jnp.dot(mt, ss, preferred_element_type=jnp.float32)
        cnt = float(N * CG)
        mean_g = gs / cnt
        var_g = gss / cnt - mean_g * mean_g
        mean_c = jax.lax.dot_general(
            mt, mean_g, dimension_numbers=(((0,), (0,)), ((), ())),
            preferred_element_type=jnp.float32)
        var_c = jax.lax.dot_general(
            mt, var_g, dimension_numbers=(((0,), (0,)), ((), ())),
            preferred_element_type=jnp.float32)
        inv = jax.lax.rsqrt(var_c + EPS_GN)
        hn = (h - mean_c) * inv * gw_ref[...] + gb_ref[...]
        return hn * jax.nn.sigmoid(hn)

    s1 = gcn_gn_silu(xn, w1_ref, b1_ref, g1w_ref, g1b_ref)
    s2 = gcn_gn_silu(s1, w2_ref, b2_ref, g2w_ref, g2b_ref)
    out_ref[0] = s2


def kernel(x, W1, b1, W2, b2, gn1_w, gn1_b, gn2_w, gn2_b):
    xc = x.reshape(B, C, N)
    col = lambda v: v.reshape(C, 1)
    full = lambda shape: pl.BlockSpec(shape, lambda b: (0,) * len(shape))

    y = pl.pallas_call(
        _gcn_block_kernel,
        grid=(B,),
        in_specs=[
            pl.BlockSpec((1, C, N), lambda b: (b, 0, 0)),
            full((C, C)), full((C, 1)),
            full((C, C)), full((C, 1)),
            full((C, 1)), full((C, 1)),
            full((C, 1)), full((C, 1)),
        ],
        out_specs=pl.BlockSpec((1, C, N), lambda b: (b, 0, 0)),
        out_shape=jax.ShapeDtypeStruct((B, C, N), jnp.float32),
        compiler_params=pltpu.CompilerParams(
            dimension_semantics=("arbitrary",)),
    )(xc, W1, col(b1), W2, col(b2),
      col(gn1_w), col(gn1_b), col(gn2_w), col(gn2_b))

    return y.reshape(B, C, H, W_)


# speedup vs baseline: 1.1948x; 1.1948x over previous
import jax
import jax.numpy as jnp
from jax.experimental import pallas as pl
from jax.experimental.pallas import tpu as pltpu

B, C, H, W_ = 8, 96, 32, 32
N = H * W_
K = 9
G = 4
CG = C // G
EPS_GN = 1e-5
NEG = -3.0e38


def _gcn_block_kernel(x_ref, w1_ref, b1_ref, w2_ref, b2_ref,
                      g1w_ref, g1b_ref, g2w_ref, g2b_ref, out_ref):
    xc = x_ref[0]

    nrm = jnp.sqrt(jnp.sum(xc * xc, axis=0, keepdims=True))
    xn = xc / jnp.maximum(nrm, 1e-12)

    sim = jax.lax.dot_general(
        xn, xn, dimension_numbers=(((0,), (0,)), ((), ())),
        preferred_element_type=jnp.float32)

    m = jnp.max(sim, axis=1, keepdims=True)
    deg = m
    for _ in range(K - 1):
        m = jnp.max(jnp.where(sim < m, sim, NEG), axis=1, keepdims=True)
        deg = deg + m
    thresh = m
    deg = deg + 1e-6

    w = jnp.where(sim >= thresh, sim, 0.0)

    c_io = jax.lax.broadcasted_iota(jnp.int32, (G, C), 1)
    g_io = jax.lax.broadcasted_iota(jnp.int32, (G, C), 0)
    mt = (c_io // CG == g_io).astype(jnp.float32)

    def gn_silu(h, gw_ref, gb_ref):
        s = jnp.sum(h, axis=0, keepdims=True)
        ss = jnp.sum(h * h, axis=0, keepdims=True)
        gs = jax.lax.dot_general(
            s, mt, dimension_numbers=(((1,), (1,)), ((), ())),
            preferred_element_type=jnp.float32)
        gss = jax.lax.dot_general(
            ss, mt, dimension_numbers=(((1,), (1,)), ((), ())),
            preferred_element_type=jnp.float32)
        cnt = float(N * CG)
        mean_g = gs / cnt
        var_g = gss / cnt - mean_g * mean_g
        mean_c = jnp.dot(mean_g, mt, preferred_element_type=jnp.float32)
        var_c = jnp.dot(var_g, mt, preferred_element_type=jnp.float32)
        inv = jax.lax.rsqrt(var_c + EPS_GN)
        hn = (h - mean_c) * inv * gw_ref[...] + gb_ref[...]
        return hn * jax.nn.sigmoid(hn)

    xt1 = jax.lax.dot_general(
        xn, w1_ref[...], dimension_numbers=(((0,), (0,)), ((), ())),
        preferred_element_type=jnp.float32)
    h1 = (jnp.dot(w, xt1, preferred_element_type=jnp.float32) / deg
          + b1_ref[...])
    s1 = gn_silu(h1, g1w_ref, g1b_ref)

    xt2 = jnp.dot(s1, w2_ref[...], preferred_element_type=jnp.float32)
    h2 = (jnp.dot(w, xt2, preferred_element_type=jnp.float32) / deg
          + b2_ref[...])
    out_ref[0] = gn_silu(h2, g2w_ref, g2b_ref)


def kernel(x, W1, b1, W2, b2, gn1_w, gn1_b, gn2_w, gn2_b):
    xc = x.reshape(B, C, N)
    vec = lambda v: v.reshape(1, C)
    full = lambda shape: pl.BlockSpec(shape, lambda b: (0,) * len(shape))

    y = pl.pallas_call(
        _gcn_block_kernel,
        grid=(B,),
        in_specs=[
            pl.BlockSpec((1, C, N), lambda b: (b, 0, 0)),
            full((C, C)), full((1, C)),
            full((C, C)), full((1, C)),
            full((1, C)), full((1, C)),
            full((1, C)), full((1, C)),
        ],
        out_specs=pl.BlockSpec((1, N, C), lambda b: (b, 0, 0)),
        out_shape=jax.ShapeDtypeStruct((B, N, C), jnp.float32),
        compiler_params=pltpu.CompilerParams(
            dimension_semantics=("arbitrary",)),
    )(xc, W1, vec(b1), W2, vec(b2),
      vec(gn1_w), vec(gn1_b), vec(gn2_w), vec(gn2_b))

    return y.transpose(0, 2, 1).reshape(B, C, H, W_)


# speedup vs baseline: 1.4952x; 1.2514x over previous
import jax
import jax.numpy as jnp
from jax.experimental import pallas as pl
from jax.experimental.pallas import tpu as pltpu

B, C, H, W_ = 8, 96, 32, 32
N = H * W_
K = 9
G = 4
CG = C // G
EPS_GN = 1e-5
NEG = -3.0e38
HALF = N // 2


def _gcn_block_kernel(x_ref, w1_ref, b1_ref, w2_ref, b2_ref,
                      g1w_ref, g1b_ref, g2w_ref, g2b_ref, out_ref):
    xf = x_ref[0]

    nrm = jnp.sqrt(jnp.sum(xf * xf, axis=1, keepdims=True))
    xn = xf / jnp.maximum(nrm, 1e-12)

    xt1 = jnp.dot(xn, w1_ref[...], preferred_element_type=jnp.float32)

    def topk_weights(rows):
        sim = jax.lax.dot_general(
            rows, xn, dimension_numbers=(((1,), (1,)), ((), ())),
            preferred_element_type=jnp.float32)
        m = jnp.max(sim, axis=1, keepdims=True)
        deg = m
        for _ in range(K - 1):
            m = jnp.max(jnp.where(sim < m, sim, NEG), axis=1, keepdims=True)
            deg = deg + m
        w = jnp.where(sim >= m, sim, 0.0)
        return w, deg + 1e-6

    wa, dega = topk_weights(xn[:HALF])
    wb, degb = topk_weights(xn[HALF:])

    c_io = jax.lax.broadcasted_iota(jnp.int32, (G, C), 1)
    g_io = jax.lax.broadcasted_iota(jnp.int32, (G, C), 0)
    mt = (c_io // CG == g_io).astype(jnp.float32)

    def aggregate(xt, b_ref):
        ha = jnp.dot(wa, xt, preferred_element_type=jnp.float32) / dega
        hb = jnp.dot(wb, xt, preferred_element_type=jnp.float32) / degb
        return jnp.concatenate([ha, hb], axis=0) + b_ref[...]

    def gn_silu(h, gw_ref, gb_ref):
        s = jnp.sum(h, axis=0, keepdims=True)
        ss = jnp.sum(h * h, axis=0, keepdims=True)
        gs = jax.lax.dot_general(
            s, mt, dimension_numbers=(((1,), (1,)), ((), ())),
            preferred_element_type=jnp.float32)
        gss = jax.lax.dot_general(
            ss, mt, dimension_numbers=(((1,), (1,)), ((), ())),
            preferred_element_type=jnp.float32)
        cnt = float(N * CG)
        mean_g = gs / cnt
        var_g = gss / cnt - mean_g * mean_g
        mean_c = jnp.dot(mean_g, mt, preferred_element_type=jnp.float32)
        var_c = jnp.dot(var_g, mt, preferred_element_type=jnp.float32)
        inv = jax.lax.rsqrt(var_c + EPS_GN)
        hn = (h - mean_c) * inv * gw_ref[...] + gb_ref[...]
        return hn * jax.nn.sigmoid(hn)

    s1 = gn_silu(aggregate(xt1, b1_ref), g1w_ref, g1b_ref)
    xt2 = jnp.dot(s1, w2_ref[...], preferred_element_type=jnp.float32)
    out_ref[0] = gn_silu(aggregate(xt2, b2_ref), g2w_ref, g2b_ref)


def kernel(x, W1, b1, W2, b2, gn1_w, gn1_b, gn2_w, gn2_b):
    xf = x.reshape(B, C, N).transpose(0, 2, 1)
    vec = lambda v: v.reshape(1, C)
    full = lambda shape: pl.BlockSpec(shape, lambda b: (0,) * len(shape))

    y = pl.pallas_call(
        _gcn_block_kernel,
        grid=(B,),
        in_specs=[
            pl.BlockSpec((1, N, C), lambda b: (b, 0, 0)),
            full((C, C)), full((1, C)),
            full((C, C)), full((1, C)),
            full((1, C)), full((1, C)),
            full((1, C)), full((1, C)),
        ],
        out_specs=pl.BlockSpec((1, N, C), lambda b: (b, 0, 0)),
        out_shape=jax.ShapeDtypeStruct((B, N, C), jnp.float32),
        compiler_params=pltpu.CompilerParams(
            dimension_semantics=("arbitrary",)),
    )(xf, W1, vec(b1), W2, vec(b2),
      vec(gn1_w), vec(gn1_b), vec(gn2_w), vec(gn2_b))

    return y.transpose(0, 2, 1).reshape(B, C, H, W_)


# speedup vs baseline: 1.5407x; 1.0304x over previous
import jax
import jax.numpy as jnp
from jax.experimental import pallas as pl
from jax.experimental.pallas import tpu as pltpu

B, C, H, W_ = 8, 96, 32, 32
N = H * W_
K = 9
G = 4
CG = C // G
EPS_GN = 1e-5
NEG = -3.0e38
QUARTER = N // 4


def _gcn_block_kernel(x_ref, w1_ref, b1_ref, w2_ref, b2_ref,
                      g1w_ref, g1b_ref, g2w_ref, g2b_ref, out_ref):
    xf = x_ref[0]

    nrm = jnp.sqrt(jnp.sum(xf * xf, axis=1, keepdims=True))
    xn = xf / jnp.maximum(nrm, 1e-12)

    xt1 = jnp.dot(xn, w1_ref[...], preferred_element_type=jnp.float32)

    def topk_weights(rows):
        sim = jax.lax.dot_general(
            rows, xn, dimension_numbers=(((1,), (1,)), ((), ())),
            preferred_element_type=jnp.float32)
        m = jnp.max(sim, axis=1, keepdims=True)
        deg = m
        for _ in range(K - 1):
            m = jnp.max(jnp.where(sim < m, sim, NEG), axis=1, keepdims=True)
            deg = deg + m
        w = jnp.where(sim >= m, sim, 0.0)
        return w, deg + 1e-6

    parts = [topk_weights(xn[i * QUARTER:(i + 1) * QUARTER])
             for i in range(4)]

    c_io = jax.lax.broadcasted_iota(jnp.int32, (G, C), 1)
    g_io = jax.lax.broadcasted_iota(jnp.int32, (G, C), 0)
    mt = (c_io // CG == g_io).astype(jnp.float32)

    def aggregate(xt, b_ref):
        hs = [jnp.dot(w, xt, preferred_element_type=jnp.float32) / deg
              for w, deg in parts]
        return jnp.concatenate(hs, axis=0) + b_ref[...]

    def gn_silu(h, gw_ref, gb_ref):
        s = jnp.sum(h, axis=0, keepdims=True)
        ss = jnp.sum(h * h, axis=0, keepdims=True)
        gs = jax.lax.dot_general(
            s, mt, dimension_numbers=(((1,), (1,)), ((), ())),
            preferred_element_type=jnp.float32)
        gss = jax.lax.dot_general(
            ss, mt, dimension_numbers=(((1,), (1,)), ((), ())),
            preferred_element_type=jnp.float32)
        cnt = float(N * CG)
        mean_g = gs / cnt
        var_g = gss / cnt - mean_g * mean_g
        mean_c = jnp.dot(mean_g, mt, preferred_element_type=jnp.float32)
        var_c = jnp.dot(var_g, mt, preferred_element_type=jnp.float32)
        inv = jax.lax.rsqrt(var_c + EPS_GN)
        hn = (h - mean_c) * inv * gw_ref[...] + gb_ref[...]
        return hn * jax.nn.sigmoid(hn)

    s1 = gn_silu(aggregate(xt1, b1_ref), g1w_ref, g1b_ref)
    xt2 = jnp.dot(s1, w2_ref[...], preferred_element_type=jnp.float32)
    out_ref[0] = gn_silu(aggregate(xt2, b2_ref), g2w_ref, g2b_ref)


def kernel(x, W1, b1, W2, b2, gn1_w, gn1_b, gn2_w, gn2_b):
    xf = x.reshape(B, C, N).transpose(0, 2, 1)
    vec = lambda v: v.reshape(1, C)
    full = lambda shape: pl.BlockSpec(shape, lambda b: (0,) * len(shape))

    y = pl.pallas_call(
        _gcn_block_kernel,
        grid=(B,),
        in_specs=[
            pl.BlockSpec((1, N, C), lambda b: (b, 0, 0)),
            full((C, C)), full((1, C)),
            full((C, C)), full((1, C)),
            full((1, C)), full((1, C)),
            full((1, C)), full((1, C)),
        ],
        out_specs=pl.BlockSpec((1, N, C), lambda b: (b, 0, 0)),
        out_shape=jax.ShapeDtypeStruct((B, N, C), jnp.float32),
        compiler_params=pltpu.CompilerParams(
            dimension_semantics=("arbitrary",)),
    )(xf, W1, vec(b1), W2, vec(b2),
      vec(gn1_w), vec(gn1_b), vec(gn2_w), vec(gn2_b))

    return y.transpose(0, 2, 1).reshape(B, C, H, W_)
